# baseline (speedup 1.0000x reference)
import functools

import jax
import jax.numpy as jnp
from jax import lax
from jax.experimental import pallas as pl
from jax.experimental.pallas import tpu as pltpu
from jax.experimental.pallas import tpu_sc as plsc

NUM_CORES = 2
NUM_SUBCORES = 16
LANES = 16
NW = NUM_CORES * NUM_SUBCORES

BATCH = 16384
DIM = 16
B_PER_W = BATCH // NW
CHUNK = 8
N_CHUNKS = B_PER_W // CHUNK
IDX_ROWS = B_PER_W // 128


def _rsqrt(x):
    xi = plsc.bitcast(x, jnp.int32)
    yi = jnp.int32(0x5F3759DF) - (xi >> 1)
    y = plsc.bitcast(yi, jnp.float32)
    half = x * jnp.float32(0.5)
    for _ in range(3):
        y = y * (jnp.float32(1.5) - half * y * y)
    return y


def _cosine_body(eidx_hbm, oidx_hbm, te_hbm, to_hbm, out_hbm,
                 eidx_v, oidx_v, ebuf, obuf, out_v,
                 esem, osem):
    wid = lax.axis_index("s") * NUM_CORES + lax.axis_index("c")
    base = wid * B_PER_W

    pltpu.sync_copy(eidx_hbm.at[pl.ds(wid * IDX_ROWS, IDX_ROWS)], eidx_v)
    pltpu.sync_copy(oidx_hbm.at[pl.ds(wid * IDX_ROWS, IDX_ROWS)], oidx_v)

    lane0 = lax.iota(jnp.int32, LANES)
    kvec0 = lane0 & 7

    def fire(c, buf):
        pos = c * CHUNK + kvec0
        ids_e = plsc.load_gather(eidx_v, [pos >> 7, pos & 127])
        ids_o = plsc.load_gather(oidx_v, [pos >> 7, pos & 127])
        ecs_v = (ids_e >> 7) << 7
        ocs_v = (ids_o >> 7) << 7
        for k in range(CHUNK):
            ecs = pl.multiple_of(
                jnp.squeeze(lax.slice(ecs_v, (k,), (k + 1,))) * 0, 128)
            ocs = pl.multiple_of(
                jnp.squeeze(lax.slice(ocs_v, (k,), (k + 1,))) * 0, 128)
            pltpu.async_copy(
                te_hbm.at[:, pl.ds(ecs, 128)], ebuf.at[buf, k], esem)
            pltpu.async_copy(
                to_hbm.at[:, pl.ds(ocs, 128)], obuf.at[buf, k], osem)

    def drain(buf):
        for k in range(CHUNK):
            pltpu.make_async_copy(
                te_hbm.at[:, pl.ds(0, 128)], ebuf.at[buf, k], esem).wait()
            pltpu.make_async_copy(
                to_hbm.at[:, pl.ds(0, 128)], obuf.at[buf, k], osem).wait()

    lane = lax.iota(jnp.int32, LANES)
    kvec = lane & 7

    def compute(c, buf):
        pos = c * CHUNK + kvec
        ids_e = plsc.load_gather(eidx_v, [pos >> 7, pos & 127])
        ids_o = plsc.load_gather(oidx_v, [pos >> 7, pos & 127])
        ecols = ids_e & 127
        ocols = ids_o & 127
        bvec = jnp.full((LANES,), buf, jnp.int32)
        dot = jnp.zeros((LANES,), jnp.float32)
        se = jnp.zeros((LANES,), jnp.float32)
        so = jnp.zeros((LANES,), jnp.float32)
        for d in range(DIM):
            dvec = jnp.full((LANES,), d, jnp.int32)
            ge = plsc.load_gather(ebuf, [bvec, kvec, dvec, ecols])
            go = plsc.load_gather(obuf, [bvec, kvec, dvec, ocols])
            dot = dot + ge * go
            se = se + ge * ge
            so = so + go * go
        out_v[pl.ds(c * CHUNK, LANES)] = dot * _rsqrt(se * so)

    fire(0, 0)

    def step(c, _):
        buf = c & 1

        @pl.when(c < N_CHUNKS - 1)
        def _():
            fire(c + 1, 1 - buf)

        drain(buf)
        compute(c, buf)
        return 0

    lax.fori_loop(0, N_CHUNKS, step, 0)

    pltpu.sync_copy(out_v.at[pl.ds(0, B_PER_W)],
                    out_hbm.at[pl.ds(base, B_PER_W)])


@functools.partial(
    pl.kernel,
    out_type=jax.ShapeDtypeStruct((BATCH,), jnp.float32),
    mesh=plsc.VectorSubcoreMesh(core_axis_name="c", subcore_axis_name="s"),
    scratch_types=[
        pltpu.VMEM((IDX_ROWS, 128), jnp.int32),
        pltpu.VMEM((IDX_ROWS, 128), jnp.int32),
        pltpu.VMEM((2, CHUNK, DIM, 128), jnp.float32),
        pltpu.VMEM((2, CHUNK, DIM, 128), jnp.float32),
        pltpu.VMEM((B_PER_W + LANES,), jnp.float32),
        pltpu.SemaphoreType.DMA,
        pltpu.SemaphoreType.DMA,
    ],
    compiler_params=pltpu.CompilerParams(
        needs_layout_passes=False,
        use_tc_tiling_on_sc=True,
    ),
)
def _cosine_kernel(*args):
    _cosine_body(*args)


def kernel(e_id, o_id, mentees, mentors):
    eidx2 = e_id.astype(jnp.int32).reshape(NW * IDX_ROWS, 128)
    oidx2 = o_id.astype(jnp.int32).reshape(NW * IDX_ROWS, 128)
    return _cosine_kernel(eidx2, oidx2, mentees.T, mentors.T)


# speedup vs baseline: 10.9950x; 10.9950x over previous
import functools

import jax
import jax.numpy as jnp
from jax import lax
from jax.experimental import pallas as pl
from jax.experimental.pallas import tpu as pltpu
from jax.experimental.pallas import tpu_sc as plsc

NUM_CORES = 2
NUM_SUBCORES = 16
LANES = 16
NW = NUM_CORES * NUM_SUBCORES

BATCH = 16384
DIM = 16
B_PER_W = BATCH // NW
CHUNK = 8
N_CHUNKS = B_PER_W // CHUNK
NBUF = 3
IDX_ROWS = B_PER_W // 128


def _rsqrt(x):
    xi = plsc.bitcast(x, jnp.int32)
    yi = jnp.int32(0x5F3759DF) - (xi >> 1)
    y = plsc.bitcast(yi, jnp.float32)
    half = x * jnp.float32(0.5)
    for _ in range(3):
        y = y * (jnp.float32(1.5) - half * y * y)
    return y


def _cosine_body(eidx_hbm, oidx_hbm, te_hbm, to_hbm, out_hbm,
                 eidx_v, oidx_v, ebuf, obuf, out_v,
                 esem, osem):
    wid = lax.axis_index("s") * NUM_CORES + lax.axis_index("c")
    base = wid * B_PER_W

    pltpu.sync_copy(eidx_hbm.at[pl.ds(wid * IDX_ROWS, IDX_ROWS)], eidx_v)
    pltpu.sync_copy(oidx_hbm.at[pl.ds(wid * IDX_ROWS, IDX_ROWS)], oidx_v)

    lane0 = lax.iota(jnp.int32, LANES)
    kvec0 = lane0 & 7

    def fire(c, buf):
        pos = c * CHUNK + kvec0
        ids_e = plsc.load_gather(eidx_v, [pos >> 7, pos & 127])
        ids_o = plsc.load_gather(oidx_v, [pos >> 7, pos & 127])
        ecs_v = (ids_e >> 7) << 7
        ocs_v = (ids_o >> 7) << 7
        for k in range(CHUNK):
            ecs = pl.multiple_of(
                jnp.squeeze(lax.slice(ecs_v, (k,), (k + 1,))), 128)
            ocs = pl.multiple_of(
                jnp.squeeze(lax.slice(ocs_v, (k,), (k + 1,))), 128)
            pltpu.async_copy(
                te_hbm.at[:, pl.ds(ecs, 128)], ebuf.at[buf, k], esem)
            pltpu.async_copy(
                to_hbm.at[:, pl.ds(ocs, 128)], obuf.at[buf, k], osem)

    def drain(buf):
        for k in range(CHUNK):
            pltpu.make_async_copy(
                te_hbm.at[:, pl.ds(0, 128)], ebuf.at[buf, k], esem).wait()
            pltpu.make_async_copy(
                to_hbm.at[:, pl.ds(0, 128)], obuf.at[buf, k], osem).wait()

    lane = lax.iota(jnp.int32, LANES)
    kvec = lane & 7

    def compute(c, buf):
        pos = c * CHUNK + kvec
        ids_e = plsc.load_gather(eidx_v, [pos >> 7, pos & 127])
        ids_o = plsc.load_gather(oidx_v, [pos >> 7, pos & 127])
        ecols = ids_e & 127
        ocols = ids_o & 127
        bvec = jnp.full((LANES,), buf, jnp.int32)
        dot = jnp.zeros((LANES,), jnp.float32)
        se = jnp.zeros((LANES,), jnp.float32)
        so = jnp.zeros((LANES,), jnp.float32)
        for d in range(DIM):
            dvec = jnp.full((LANES,), d, jnp.int32)
            ge = plsc.load_gather(ebuf, [bvec, kvec, dvec, ecols])
            go = plsc.load_gather(obuf, [bvec, kvec, dvec, ocols])
            dot = dot + ge * go
            se = se + ge * ge
            so = so + go * go
        out_v[pl.ds(c * CHUNK, LANES)] = dot * _rsqrt(se * so)

    for p in range(NBUF - 1):
        fire(p, p)

    def step(c, _):
        buf = c % NBUF

        @pl.when(c < N_CHUNKS - (NBUF - 1))
        def _():
            fire(c + NBUF - 1, (c + NBUF - 1) % NBUF)

        drain(buf)
        compute(c, buf)
        return 0

    lax.fori_loop(0, N_CHUNKS, step, 0)

    pltpu.sync_copy(out_v.at[pl.ds(0, B_PER_W)],
                    out_hbm.at[pl.ds(base, B_PER_W)])


@functools.partial(
    pl.kernel,
    out_type=jax.ShapeDtypeStruct((BATCH,), jnp.float32),
    mesh=plsc.VectorSubcoreMesh(core_axis_name="c", subcore_axis_name="s"),
    scratch_types=[
        pltpu.VMEM((IDX_ROWS, 128), jnp.int32),
        pltpu.VMEM((IDX_ROWS, 128), jnp.int32),
        pltpu.VMEM((NBUF, CHUNK, DIM, 128), jnp.float32),
        pltpu.VMEM((NBUF, CHUNK, DIM, 128), jnp.float32),
        pltpu.VMEM((B_PER_W + LANES,), jnp.float32),
        pltpu.SemaphoreType.DMA,
        pltpu.SemaphoreType.DMA,
    ],
    compiler_params=pltpu.CompilerParams(
        needs_layout_passes=False,
        use_tc_tiling_on_sc=True,
    ),
)
def _cosine_kernel(*args):
    _cosine_body(*args)


def kernel(e_id, o_id, mentees, mentors):
    eidx2 = e_id.astype(jnp.int32).reshape(NW * IDX_ROWS, 128)
    oidx2 = o_id.astype(jnp.int32).reshape(NW * IDX_ROWS, 128)
    return _cosine_kernel(eidx2, oidx2, mentees.T, mentors.T)


# speedup vs baseline: 11.0462x; 1.0047x over previous
import functools

import jax
import jax.numpy as jnp
from jax import lax
from jax.experimental import pallas as pl
from jax.experimental.pallas import tpu as pltpu
from jax.experimental.pallas import tpu_sc as plsc

NUM_CORES = 2
NUM_SUBCORES = 16
LANES = 16
NW = NUM_CORES * NUM_SUBCORES

BATCH = 16384
DIM = 16
B_PER_W = BATCH // NW
CHUNK = 4
N_CHUNKS = B_PER_W // CHUNK
NBUF = 6
IDX_ROWS = B_PER_W // 128


def _rsqrt(x):
    xi = plsc.bitcast(x, jnp.int32)
    yi = jnp.int32(0x5F3759DF) - (xi >> 1)
    y = plsc.bitcast(yi, jnp.float32)
    half = x * jnp.float32(0.5)
    for _ in range(3):
        y = y * (jnp.float32(1.5) - half * y * y)
    return y


def _cosine_body(eidx_hbm, oidx_hbm, te_hbm, to_hbm, out_hbm,
                 eidx_v, oidx_v, ebuf, obuf, out_v,
                 esem, osem):
    wid = lax.axis_index("s") * NUM_CORES + lax.axis_index("c")
    base = wid * B_PER_W

    pltpu.sync_copy(eidx_hbm.at[pl.ds(wid * IDX_ROWS, IDX_ROWS)], eidx_v)
    pltpu.sync_copy(oidx_hbm.at[pl.ds(wid * IDX_ROWS, IDX_ROWS)], oidx_v)

    lane0 = lax.iota(jnp.int32, LANES)
    kvec0 = lane0 & (CHUNK - 1)

    def fire(c, buf):
        pos = c * CHUNK + kvec0
        ids_e = plsc.load_gather(eidx_v, [pos >> 7, pos & 127])
        ids_o = plsc.load_gather(oidx_v, [pos >> 7, pos & 127])
        ecs_v = (ids_e >> 7) << 7
        ocs_v = (ids_o >> 7) << 7
        for k in range(CHUNK):
            ecs = pl.multiple_of(
                jnp.squeeze(lax.slice(ecs_v, (k,), (k + 1,))), 128)
            ocs = pl.multiple_of(
                jnp.squeeze(lax.slice(ocs_v, (k,), (k + 1,))), 128)
            pltpu.async_copy(
                te_hbm.at[:, pl.ds(ecs, 128)], ebuf.at[buf, k], esem)
            pltpu.async_copy(
                to_hbm.at[:, pl.ds(ocs, 128)], obuf.at[buf, k], osem)

    def drain(buf):
        for k in range(CHUNK):
            pltpu.make_async_copy(
                te_hbm.at[:, pl.ds(0, 128)], ebuf.at[buf, k], esem).wait()
            pltpu.make_async_copy(
                to_hbm.at[:, pl.ds(0, 128)], obuf.at[buf, k], osem).wait()

    lane = lax.iota(jnp.int32, LANES)
    kvec = lane & (CHUNK - 1)

    def compute(c, buf):
        pos = c * CHUNK + kvec
        ids_e = plsc.load_gather(eidx_v, [pos >> 7, pos & 127])
        ids_o = plsc.load_gather(oidx_v, [pos >> 7, pos & 127])
        ecols = ids_e & 127
        ocols = ids_o & 127
        bvec = jnp.full((LANES,), buf, jnp.int32)
        dot = jnp.zeros((LANES,), jnp.float32)
        se = jnp.zeros((LANES,), jnp.float32)
        so = jnp.zeros((LANES,), jnp.float32)
        for d in range(DIM):
            dvec = jnp.full((LANES,), d, jnp.int32)
            ge = plsc.load_gather(ebuf, [bvec, kvec, dvec, ecols])
            go = plsc.load_gather(obuf, [bvec, kvec, dvec, ocols])
            dot = dot + ge * go
            se = se + ge * ge
            so = so + go * go
        out_v[pl.ds(c * CHUNK, LANES)] = dot * _rsqrt(se * so)

    for p in range(NBUF - 1):
        fire(p, p)

    def step(c, _):
        buf = c % NBUF

        @pl.when(c < N_CHUNKS - (NBUF - 1))
        def _():
            fire(c + NBUF - 1, (c + NBUF - 1) % NBUF)

        drain(buf)
        compute(c, buf)
        return 0

    lax.fori_loop(0, N_CHUNKS, step, 0)

    pltpu.sync_copy(out_v.at[pl.ds(0, B_PER_W)],
                    out_hbm.at[pl.ds(base, B_PER_W)])


@functools.partial(
    pl.kernel,
    out_type=jax.ShapeDtypeStruct((BATCH,), jnp.float32),
    mesh=plsc.VectorSubcoreMesh(core_axis_name="c", subcore_axis_name="s"),
    scratch_types=[
        pltpu.VMEM((IDX_ROWS, 128), jnp.int32),
        pltpu.VMEM((IDX_ROWS, 128), jnp.int32),
        pltpu.VMEM((NBUF, CHUNK, DIM, 128), jnp.float32),
        pltpu.VMEM((NBUF, CHUNK, DIM, 128), jnp.float32),
        pltpu.VMEM((B_PER_W + LANES,), jnp.float32),
        pltpu.SemaphoreType.DMA,
        pltpu.SemaphoreType.DMA,
    ],
    compiler_params=pltpu.CompilerParams(
        needs_layout_passes=False,
        use_tc_tiling_on_sc=True,
    ),
)
def _cosine_kernel(*args):
    _cosine_body(*args)


def kernel(e_id, o_id, mentees, mentors):
    eidx2 = e_id.astype(jnp.int32).reshape(NW * IDX_ROWS, 128)
    oidx2 = o_id.astype(jnp.int32).reshape(NW * IDX_ROWS, 128)
    return _cosine_kernel(eidx2, oidx2, mentees.T, mentors.T)
